# Initial kernel scaffold; baseline (speedup 1.0000x reference)
import jax
import jax.numpy as jnp
from jax.experimental import pallas as pl


def kernel(x, table):
    raise NotImplementedError("write your pallas kernel here")


# speedup vs baseline: 3.3778x; 3.3778x over previous
import functools

import jax
import jax.numpy as jnp
from jax import lax
from jax.experimental import pallas as pl
from jax.experimental.pallas import tpu as pltpu
from jax.experimental.pallas import tpu_sc as plsc

NC, NS = 2, 16
NW = NC * NS
BATCH = 4096
NFIELD = 26
FIELD_SIZE = 1000
D = 64
B = BATCH * NFIELD
BPW = B // NW
LANES = 16
NVEC = BPW // LANES
CHUNK = 128
NCHUNK = BPW // CHUNK

_mesh = plsc.VectorSubcoreMesh(
    core_axis_name="c", subcore_axis_name="s", num_cores=NC, num_subcores=NS
)


@functools.partial(
    pl.kernel,
    out_type=jax.ShapeDtypeStruct((B, D), jnp.float32),
    mesh=_mesh,
    scratch_types=[
        pltpu.VMEM((BPW,), jnp.int32),
        pltpu.VMEM((CHUNK, D), jnp.float32),
        pltpu.SemaphoreType.DMA,
    ],
    compiler_params=pltpu.CompilerParams(use_tc_tiling_on_sc=False),
)
def _gather_kernel(x_hbm, table_hbm, out_hbm, idx_v, buf, sem):
    wid = lax.axis_index("s") * NC + lax.axis_index("c")
    base = wid * BPW

    pltpu.sync_copy(x_hbm.at[pl.ds(base, BPW)], idx_v)
    lane = lax.iota(jnp.int32, LANES)

    def add_offsets(j, carry):
        s = pl.ds(j * LANES, LANES)
        field = lax.rem(j * LANES + lane, NFIELD)
        idx_v[s] = idx_v[s] + field * FIELD_SIZE
        return carry

    lax.fori_loop(0, NVEC, add_offsets, 0)

    def do_chunk(j, carry):
        src = table_hbm.at[idx_v.at[pl.ds(j * CHUNK, CHUNK)]]
        pltpu.async_copy(src, buf, sem).wait()
        pltpu.sync_copy(buf, out_hbm.at[pl.ds(base + j * CHUNK, CHUNK)])
        return carry

    lax.fori_loop(0, NCHUNK, do_chunk, 0)


def kernel(x, table):
    out = _gather_kernel(x.reshape(B), table)
    return out.reshape(BATCH, NFIELD, D)


# speedup vs baseline: 3.8046x; 1.1264x over previous
import functools

import jax
import jax.numpy as jnp
from jax import lax
from jax.experimental import pallas as pl
from jax.experimental.pallas import tpu as pltpu
from jax.experimental.pallas import tpu_sc as plsc

NC, NS = 2, 16
NW = NC * NS
BATCH = 4096
NFIELD = 26
FIELD_SIZE = 1000
D = 64
B = BATCH * NFIELD
BPW = B // NW
LANES = 16
NVEC = BPW // LANES
SUB = 104
NSUB = 8
BIG = SUB * NSUB
NBIG = BPW // BIG

_mesh = plsc.VectorSubcoreMesh(
    core_axis_name="c", subcore_axis_name="s", num_cores=NC, num_subcores=NS
)


@functools.partial(
    pl.kernel,
    out_type=jax.ShapeDtypeStruct((B, D), jnp.float32),
    mesh=_mesh,
    scratch_types=[
        pltpu.VMEM((BPW,), jnp.int32),
        pltpu.VMEM((BIG, D), jnp.float32),
        pltpu.VMEM((BIG, D), jnp.float32),
        pltpu.SemaphoreType.DMA,
        pltpu.SemaphoreType.DMA,
        pltpu.SemaphoreType.DMA,
        pltpu.SemaphoreType.DMA,
    ],
    compiler_params=pltpu.CompilerParams(use_tc_tiling_on_sc=False),
)
def _gather_kernel(x_hbm, table_hbm, out_hbm, idx_v, buf0, buf1, gs0, gs1, os0, os1):
    wid = lax.axis_index("s") * NC + lax.axis_index("c")
    base = wid * BPW

    pltpu.sync_copy(x_hbm.at[pl.ds(base, BPW)], idx_v)
    lane = lax.iota(jnp.int32, LANES)
    wrap = NFIELD * FIELD_SIZE

    def add_offsets(j, off):
        s = pl.ds(j * LANES, LANES)
        idx_v[s] = idx_v[s] + off
        off = off + LANES * FIELD_SIZE
        return jnp.where(off >= wrap, off - wrap, off)

    lax.fori_loop(0, NVEC, add_offsets, lane * FIELD_SIZE)

    bufs, gsems, osems = (buf0, buf1), (gs0, gs1), (os0, os1)
    outh = [None, None]
    for t in range(NBIG):
        b = t % 2
        if outh[b] is not None:
            outh[b].wait()
        hs = []
        for i in range(NSUB):
            r0 = t * BIG + i * SUB
            hs.append(pltpu.async_copy(
                table_hbm.at[idx_v.at[pl.ds(r0, SUB)]],
                bufs[b].at[pl.ds(i * SUB, SUB)],
                gsems[b]))
        for h in hs:
            h.wait()
        outh[b] = pltpu.async_copy(
            bufs[b], out_hbm.at[pl.ds(base + t * BIG, BIG)], osems[b])
    outh[0].wait()
    outh[1].wait()


def kernel(x, table):
    out = _gather_kernel(x.reshape(B), table)
    return out.reshape(BATCH, NFIELD, D)


# speedup vs baseline: 5.0048x; 1.3154x over previous
import functools

import jax
import jax.numpy as jnp
from jax import lax
from jax.experimental import pallas as pl
from jax.experimental.pallas import tpu as pltpu
from jax.experimental.pallas import tpu_sc as plsc

NC, NS = 2, 16
BATCH = 4096
NFIELD = 26
FIELD_SIZE = 1000
D = 64
DG = 8
NR = D // DG
BH = BATCH // 2
NFW = NFIELD // 2
LANES = 16
NCHUNK = BH // LANES

_mesh = plsc.VectorSubcoreMesh(
    core_axis_name="c", subcore_axis_name="s", num_cores=NC, num_subcores=NS
)


@functools.partial(
    pl.kernel,
    out_type=jax.ShapeDtypeStruct((NFIELD, NR, BATCH // 128, DG, 128), jnp.float32),
    mesh=_mesh,
    scratch_types=[
        pltpu.VMEM((NFIELD, BH), jnp.int32),
        pltpu.VMEM((DG, FIELD_SIZE), jnp.float32),
        pltpu.VMEM((DG, FIELD_SIZE), jnp.float32),
        pltpu.VMEM((BH // 128, DG, 128), jnp.float32),
        pltpu.VMEM((BH // 128, DG, 128), jnp.float32),
        pltpu.SemaphoreType.DMA,
        pltpu.SemaphoreType.DMA,
        pltpu.SemaphoreType.DMA,
        pltpu.SemaphoreType.DMA,
    ],
    compiler_params=pltpu.CompilerParams(
        use_tc_tiling_on_sc=False, needs_layout_passes=False
    ),
)
def _embed_kernel(tt_hbm, xt_hbm, out_hbm, xb, tb0, tb1, ob0, ob1, ts0, ts1, os0, os1):
    wid = lax.axis_index("s") * NC + lax.axis_index("c")
    r = wid // 4
    h = (wid // 2) % 2
    fp = wid % 2

    pltpu.sync_copy(xt_hbm.at[:, pl.ds(BH * h, BH)], xb)

    tbufs, tsems = (tb0, tb1), (ts0, ts1)
    obufs, osems = (ob0, ob1), (os0, os1)
    srow = [jnp.zeros((LANES,), jnp.int32) + s for s in range(DG)]

    th = [None, None]
    oh = [None, None]
    th[0] = pltpu.async_copy(
        tt_hbm.at[pl.ds(DG * r, DG), pl.ds(FIELD_SIZE * fp, FIELD_SIZE)],
        tbufs[0], tsems[0])
    for k in range(NFW):
        b = k % 2
        f = fp + 2 * k
        if k + 1 < NFW:
            nb = (k + 1) % 2
            th[nb] = pltpu.async_copy(
                tt_hbm.at[pl.ds(DG * r, DG),
                          pl.ds(FIELD_SIZE * (f + 2), FIELD_SIZE)],
                tbufs[nb], tsems[nb])
        th[b].wait()
        if oh[b] is not None:
            oh[b].wait()

        tb, ob = tbufs[b], obufs[b]

        def chunk(j, carry):
            cvec = xb[f, pl.ds(j * LANES, LANES)]
            c = j // (128 // LANES)
            l0 = (j % (128 // LANES)) * LANES
            for s in range(DG):
                ob[c, s, pl.ds(l0, LANES)] = plsc.load_gather(tb, [srow[s], cvec])
            return carry

        lax.fori_loop(0, NCHUNK, chunk, 0)

        oh[b] = pltpu.async_copy(
            ob, out_hbm.at[f, r].at[pl.ds((BH // 128) * h, BH // 128)],
            osems[b])
    oh[0].wait()
    oh[1].wait()


def kernel(x, table):
    out5 = _embed_kernel(table.T, x.T)
    return jnp.transpose(out5, (2, 4, 0, 1, 3)).reshape(BATCH, NFIELD, D)


# speedup vs baseline: 9.1921x; 1.8367x over previous
import functools

import jax
import jax.numpy as jnp
from jax import lax
from jax.experimental import pallas as pl
from jax.experimental.pallas import tpu as pltpu
from jax.experimental.pallas import tpu_sc as plsc

NC, NS = 2, 16
BATCH = 4096
NFIELD = 26
FIELD_SIZE = 1000
D = 64
DG = 8
NR = D // DG
BH = BATCH // 2
NFW = NFIELD // 2
LANES = 16
NCHUNK = BH // LANES

_mesh = plsc.VectorSubcoreMesh(
    core_axis_name="c", subcore_axis_name="s", num_cores=NC, num_subcores=NS
)


@functools.partial(
    pl.kernel,
    out_type=jax.ShapeDtypeStruct((NFIELD, NR, BATCH // 128, DG, 128), jnp.float32),
    mesh=_mesh,
    scratch_types=[
        pltpu.VMEM((NFIELD, BH), jnp.int32),
        pltpu.VMEM((DG, FIELD_SIZE), jnp.float32),
        pltpu.VMEM((DG, FIELD_SIZE), jnp.float32),
        pltpu.VMEM((BH // 128, DG, 128), jnp.float32),
        pltpu.VMEM((BH // 128, DG, 128), jnp.float32),
        pltpu.SemaphoreType.DMA,
        pltpu.SemaphoreType.DMA,
        pltpu.SemaphoreType.DMA,
        pltpu.SemaphoreType.DMA,
    ],
    compiler_params=pltpu.CompilerParams(
        use_tc_tiling_on_sc=False, needs_layout_passes=False
    ),
)
def _embed_kernel(tt_hbm, xt_hbm, out_hbm, xb, tb0, tb1, ob0, ob1, ts0, ts1, os0, os1):
    wid = lax.axis_index("s") * NC + lax.axis_index("c")
    r = wid // 4
    h = (wid // 2) % 2
    fp = wid % 2

    pltpu.sync_copy(xt_hbm.at[:, pl.ds(BH * h, BH)], xb)

    tbufs, tsems = (tb0, tb1), (ts0, ts1)
    obufs, osems = (ob0, ob1), (os0, os1)
    srow = [jnp.zeros((LANES,), jnp.int32) + s for s in range(DG)]

    th = [None, None]
    oh = [None, None]
    th[0] = pltpu.async_copy(
        tt_hbm.at[pl.ds(DG * r, DG), pl.ds(FIELD_SIZE * fp, FIELD_SIZE)],
        tbufs[0], tsems[0])
    for k in range(NFW):
        b = k % 2
        f = fp + 2 * k
        if k + 1 < NFW:
            nb = (k + 1) % 2
            th[nb] = pltpu.async_copy(
                tt_hbm.at[pl.ds(DG * r, DG),
                          pl.ds(FIELD_SIZE * (f + 2), FIELD_SIZE)],
                tbufs[nb], tsems[nb])
        th[b].wait()
        if oh[b] is not None:
            oh[b].wait()

        tb, ob = tbufs[b], obufs[b]

        @plsc.parallel_loop(0, NCHUNK, unroll=4)
        def chunk(j):
            cvec = xb[f, pl.ds(j * LANES, LANES)]
            c = j // (128 // LANES)
            l0 = (j % (128 // LANES)) * LANES
            for s in range(DG):
                ob[c, s, pl.ds(l0, LANES)] = plsc.load_gather(tb, [srow[s], cvec])

        oh[b] = pltpu.async_copy(
            ob, out_hbm.at[f, r].at[pl.ds((BH // 128) * h, BH // 128)],
            osems[b])
    oh[0].wait()
    oh[1].wait()


def kernel(x, table):
    out5 = _embed_kernel(table.T, x.T)
    return jnp.transpose(out5, (2, 4, 0, 1, 3)).reshape(BATCH, NFIELD, D)
